# baseline (speedup 1.0000x reference)
import jax
import jax.numpy as jnp
from jax.experimental import pallas as pl
from jax.experimental.pallas import tpu as pltpu


def _copy_body(k_ref, v_ref, ok_ref, ov_ref, sem):
    ck = pltpu.make_async_copy(k_ref, ok_ref, sem.at[0])
    cv = pltpu.make_async_copy(v_ref, ov_ref, sem.at[1])
    ck.start()
    cv.start()
    ck.wait()
    cv.wait()


def kernel(new_k, new_v):
    out_shape = (
        jax.ShapeDtypeStruct(new_k.shape, new_k.dtype),
        jax.ShapeDtypeStruct(new_v.shape, new_v.dtype),
    )
    return pl.pallas_call(
        _copy_body,
        in_specs=[
            pl.BlockSpec(memory_space=pl.ANY),
            pl.BlockSpec(memory_space=pl.ANY),
        ],
        out_specs=[
            pl.BlockSpec(memory_space=pl.ANY),
            pl.BlockSpec(memory_space=pl.ANY),
        ],
        out_shape=out_shape,
        scratch_shapes=[pltpu.SemaphoreType.DMA((2,))],
    )(new_k, new_v)


# speedup vs baseline: 11.3658x; 11.3658x over previous
import jax
import jax.numpy as jnp
from jax.experimental import pallas as pl
from jax.experimental.pallas import tpu as pltpu

_COLS = 2048
_BLOCK_ROWS = 512


def _copy_body(k_ref, v_ref, ok_ref, ov_ref):
    ok_ref[...] = k_ref[...]
    ov_ref[...] = v_ref[...]


def kernel(new_k, new_v):
    shape = new_k.shape
    total = new_k.size
    rows = total // _COLS
    k2 = new_k.reshape(rows, _COLS)
    v2 = new_v.reshape(rows, _COLS)
    n_blocks = rows // _BLOCK_ROWS

    spec = pl.BlockSpec((_BLOCK_ROWS, _COLS), lambda i: (i, 0))
    out2 = pl.pallas_call(
        _copy_body,
        grid=(n_blocks,),
        in_specs=[spec, spec],
        out_specs=[spec, spec],
        out_shape=(
            jax.ShapeDtypeStruct((rows, _COLS), new_k.dtype),
            jax.ShapeDtypeStruct((rows, _COLS), new_v.dtype),
        ),
        compiler_params=pltpu.CompilerParams(
            dimension_semantics=("parallel",),
        ),
    )(k2, v2)
    return (out2[0].reshape(shape), out2[1].reshape(shape))
